# baseline (speedup 1.0000x reference)
import functools

import jax
import jax.numpy as jnp
import numpy as np
from jax import lax
from jax.experimental import pallas as pl
from jax.experimental.pallas import tpu as pltpu
from jax.experimental.pallas import tpu_sc as plsc

D_MODEL = 64
SEQ = 50
NC, NS = 2, 16
NW = NC * NS
GATHER = 100
CHUNK = 200
LANES = 16


def _pos_encoding(max_len, d_model):
    position = jnp.arange(max_len, dtype=jnp.float32)[:, None]
    div_term = jnp.exp(
        jnp.arange(0, d_model, 2, dtype=jnp.float32) * (-np.log(10000.0) / d_model)
    )
    pe = jnp.zeros((max_len, d_model), dtype=jnp.float32)
    pe = pe.at[:, 0::2].set(jnp.sin(position * div_term))
    pe = pe.at[:, 1::2].set(jnp.cos(position * div_term))
    return pe


@functools.partial(jax.jit, static_argnames=("batch", "seq"))
def _embed(idx3, pe2, table, batch, seq):
    b_total = batch * seq
    bpw = b_total // NW
    nchunks = bpw // CHUNK

    mesh = plsc.VectorSubcoreMesh(
        core_axis_name="c", subcore_axis_name="s", num_cores=NC, num_subcores=NS
    )

    @functools.partial(
        pl.kernel,
        out_type=jax.ShapeDtypeStruct((b_total, D_MODEL), jnp.float32),
        mesh=mesh,
        compiler_params=pltpu.CompilerParams(use_tc_tiling_on_sc=False),
        scratch_types=[
            pltpu.VMEM((2 * nchunks, GATHER), jnp.int32),
            pltpu.VMEM((CHUNK, D_MODEL), jnp.float32),
            pltpu.VMEM((CHUNK, D_MODEL), jnp.float32),
            pltpu.SemaphoreType.DMA,
        ],
    )
    def body(idx_hbm, pe_hbm, table_hbm, out_hbm, idx_v, pe_v, buf, sem):
        wid = lax.axis_index("s") * NC + lax.axis_index("c")
        pltpu.sync_copy(idx_hbm.at[wid], idx_v)
        pltpu.sync_copy(pe_hbm, pe_v)
        base = wid * bpw

        def chunk_body(c, _):
            lo = pltpu.async_copy(
                table_hbm.at[idx_v.at[2 * c]], buf.at[pl.ds(0, GATHER)], sem
            )
            hi = pltpu.async_copy(
                table_hbm.at[idx_v.at[2 * c + 1]], buf.at[pl.ds(GATHER, GATHER)], sem
            )
            lo.wait()
            hi.wait()

            def row_body(i, _):
                for j in range(D_MODEL // LANES):
                    sl = pl.ds(j * LANES, LANES)
                    buf[i, sl] = buf[i, sl] + pe_v[i, sl]
                return 0

            lax.fori_loop(0, CHUNK, row_body, 0)
            pltpu.sync_copy(buf, out_hbm.at[pl.ds(base + c * CHUNK, CHUNK)])
            return 0

        lax.fori_loop(0, nchunks, chunk_body, 0)

    return body(idx3, pe2, table)


def kernel(x, W):
    batch, seq = x.shape
    pe = _pos_encoding(seq, D_MODEL)
    pe2 = jnp.tile(pe, (CHUNK // seq, 1))
    idx3 = x.reshape(NW, -1, GATHER)
    out = _embed(idx3, pe2, W, batch, seq)
    return out.reshape(batch, seq, D_MODEL)


# speedup vs baseline: 1.0676x; 1.0676x over previous
import functools

import jax
import jax.numpy as jnp
import numpy as np
from jax import lax
from jax.experimental import pallas as pl
from jax.experimental.pallas import tpu as pltpu
from jax.experimental.pallas import tpu_sc as plsc

D_MODEL = 64
SEQ = 50
NC, NS = 2, 16
NW = NC * NS
GATHER = 100
CHUNK = 200
RING = 4
OBUF = 2
LANES = 16
REPS = CHUNK // SEQ


def _pos_encoding(max_len, d_model):
    position = jnp.arange(max_len, dtype=jnp.float32)[:, None]
    div_term = jnp.exp(
        jnp.arange(0, d_model, 2, dtype=jnp.float32) * (-np.log(10000.0) / d_model)
    )
    pe = jnp.zeros((max_len, d_model), dtype=jnp.float32)
    pe = pe.at[:, 0::2].set(jnp.sin(position * div_term))
    pe = pe.at[:, 1::2].set(jnp.cos(position * div_term))
    return pe


@functools.partial(jax.jit, static_argnames=("batch", "seq"))
def _embed(idx3, pe, table, batch, seq):
    b_total = batch * seq
    bpw = b_total // NW
    nchunks = bpw // CHUNK
    ng = nchunks // RING

    mesh = plsc.VectorSubcoreMesh(
        core_axis_name="c", subcore_axis_name="s", num_cores=NC, num_subcores=NS
    )

    @functools.partial(
        pl.kernel,
        out_type=jax.ShapeDtypeStruct((b_total, D_MODEL), jnp.float32),
        mesh=mesh,
        compiler_params=pltpu.CompilerParams(use_tc_tiling_on_sc=False),
        scratch_types=[
            pltpu.VMEM((2 * nchunks, GATHER), jnp.int32),
            pltpu.VMEM((SEQ, D_MODEL), jnp.float32),
        ]
        + [pltpu.VMEM((CHUNK, D_MODEL), jnp.float32) for _ in range(RING)]
        + [pltpu.VMEM((CHUNK, D_MODEL), jnp.float32) for _ in range(OBUF)]
        + [pltpu.SemaphoreType.DMA for _ in range(RING + OBUF)],
    )
    def body(
        idx_hbm, pe_hbm, table_hbm, out_hbm, idx_v, pe_v,
        g0, g1, g2, g3, o0, o1, gs0, gs1, gs2, gs3, os0, os1,
    ):
        gbuf = [g0, g1, g2, g3]
        obuf = [o0, o1]
        gsem = [gs0, gs1, gs2, gs3]
        osem = [os0, os1]
        wid = lax.axis_index("s") * NC + lax.axis_index("c")
        pltpu.sync_copy(idx_hbm.at[wid], idx_v)
        pltpu.sync_copy(pe_hbm, pe_v)
        base = wid * bpw

        def start_gather(c, b):
            pltpu.async_copy(
                table_hbm.at[idx_v.at[2 * c]], gbuf[b].at[pl.ds(0, GATHER)], gsem[b]
            )
            pltpu.async_copy(
                table_hbm.at[idx_v.at[2 * c + 1]],
                gbuf[b].at[pl.ds(GATHER, GATHER)],
                gsem[b],
            )

        def wait_gather(b):
            pltpu.make_async_copy(
                table_hbm.at[pl.ds(0, CHUNK)], gbuf[b], gsem[b]
            ).wait()

        def wait_store(ob):
            pltpu.make_async_copy(
                obuf[ob], out_hbm.at[pl.ds(base, CHUNK)], osem[ob]
            ).wait()

        for b in range(RING):
            start_gather(b, b)

        def g_body(g, _):
            for b in range(RING):
                c = RING * g + b
                ob = b % OBUF
                wait_gather(b)

                def l_body(l, _):
                    for j in range(D_MODEL // LANES):
                        sl = pl.ds(j * LANES, LANES)
                        pe_vec = pe_v[l, sl]
                        for rep in range(REPS):
                            r = l + rep * SEQ
                            obuf[ob][r, sl] = gbuf[b][r, sl] + pe_vec
                    return 0

                lax.fori_loop(0, SEQ, l_body, 0)

                @pl.when(g < ng - 1)
                def _():
                    start_gather(c + RING, b)

                if b >= OBUF:
                    wait_store(ob)
                else:

                    @pl.when(g > 0)
                    def _():
                        wait_store(ob)

                pltpu.async_copy(
                    obuf[ob], out_hbm.at[pl.ds(base + c * CHUNK, CHUNK)], osem[ob]
                )
            return 0

        lax.fori_loop(0, ng, g_body, 0)
        wait_store(0)
        wait_store(1)

    return body(idx3, pe, table)


def kernel(x, W):
    batch, seq = x.shape
    pe = _pos_encoding(seq, D_MODEL)
    idx3 = x.reshape(NW, -1, GATHER)
    out = _embed(idx3, pe, W, batch, seq)
    return out.reshape(batch, seq, D_MODEL)
